# baseline (speedup 1.0000x reference)
import functools

import jax
import jax.numpy as jnp
from jax import lax
from jax.experimental import pallas as pl
from jax.experimental.pallas import tpu as pltpu
from jax.experimental.pallas import tpu_sc as plsc

NUM_EMB = 100000
DIM = 64
WIDE = 128
RANK = 8
SCALE = 2.0


FUSE_ROWS = 2000


def _fuse_body(a_ref, emb_ref, b_ref, out_ref):
    delta = lax.dot_general(
        a_ref[...], b_ref[...],
        dimension_numbers=(((1,), (1,)), ((), ())),
        preferred_element_type=jnp.float32,
    )
    out_ref[:, 0:DIM] = emb_ref[...] + SCALE * delta
    out_ref[:, DIM:WIDE] = jnp.zeros((FUSE_ROWS, WIDE - DIM), jnp.float32)


def _fused_table(emb_weight, lora_At, lora_B):
    grid = NUM_EMB // FUSE_ROWS
    return pl.pallas_call(
        _fuse_body,
        grid=(grid,),
        in_specs=[
            pl.BlockSpec((FUSE_ROWS, RANK), lambda i: (i, 0)),
            pl.BlockSpec((FUSE_ROWS, DIM), lambda i: (i, 0)),
            pl.BlockSpec((DIM, RANK), lambda i: (0, 0)),
        ],
        out_specs=pl.BlockSpec((FUSE_ROWS, WIDE), lambda i: (i, 0)),
        out_shape=jax.ShapeDtypeStruct((NUM_EMB, WIDE), jnp.float32),
    )(lora_At, emb_weight, lora_B)


NUM_CORES = 2
NUM_SUBCORES = 16
NW = NUM_CORES * NUM_SUBCORES
BATCH = 4096
SEQ = 200
B_TOTAL = BATCH * SEQ
B_PER_W = B_TOTAL // NW
BATCH_PER_W = BATCH // NW
CHUNK = 40
GROUP = 5
N_FLUSH = B_PER_W // (CHUNK * GROUP)


def _gather_body(table_hbm, idx_hbm, out_hbm, idx_v, rows_v, sv, sem):
    wid = lax.axis_index("s") * NUM_CORES + lax.axis_index("c")
    pltpu.sync_copy(idx_hbm.at[pl.ds(wid * B_PER_W, B_PER_W)], idx_v)
    b_base = wid * BATCH_PER_W

    def per_batch(g, carry):
        copies = []
        for j in range(GROUP):
            c = g * GROUP + j
            copies.append(
                pltpu.async_copy(
                    table_hbm.at[idx_v.at[pl.ds(c * CHUNK, CHUNK)]],
                    rows_v.at[pl.ds(j * CHUNK, CHUNK)],
                    sem,
                )
            )
        for cp in copies:
            cp.wait()

        def pack_row(r, c2):
            for k in range(DIM // 16):
                sv[r, pl.ds(k * 16, 16)] = rows_v[r, pl.ds(k * 16, 16)]
            return c2

        lax.fori_loop(0, SEQ, pack_row, 0)
        pltpu.sync_copy(sv, out_hbm.at[b_base + g])
        return carry

    lax.fori_loop(0, BATCH_PER_W, per_batch, 0)


def _gather(table, idx):
    mesh = plsc.VectorSubcoreMesh(core_axis_name="c", subcore_axis_name="s")
    run = functools.partial(
        pl.kernel,
        mesh=mesh,
        out_type=jax.ShapeDtypeStruct((BATCH, SEQ, DIM), jnp.float32),
        scratch_types=[
            pltpu.VMEM((B_PER_W,), jnp.int32),
            pltpu.VMEM((SEQ, WIDE), jnp.float32),
            pltpu.VMEM((SEQ, DIM), jnp.float32),
            pltpu.SemaphoreType.DMA,
        ],
    )(_gather_body)
    return run(table, idx)


def kernel(x, emb_weight, lora_A, lora_B):
    table = _fused_table(emb_weight, lora_A.T, lora_B)
    idx = x.reshape(B_TOTAL).astype(jnp.int32)
    return _gather(table, idx)


# speedup vs baseline: 1.0212x; 1.0212x over previous
import functools

import jax
import jax.numpy as jnp
from jax import lax
from jax.experimental import pallas as pl
from jax.experimental.pallas import tpu as pltpu
from jax.experimental.pallas import tpu_sc as plsc

NUM_EMB = 100000
DIM = 64
WIDE = 128
RANK = 8
SCALE = 2.0


FUSE_ROWS = 2000


def _fuse_body(a_ref, emb_ref, b_ref, out_ref):
    delta = lax.dot_general(
        a_ref[...], b_ref[...],
        dimension_numbers=(((1,), (1,)), ((), ())),
        preferred_element_type=jnp.float32,
    )
    out_ref[:, 0:DIM] = emb_ref[...] + SCALE * delta
    out_ref[:, DIM:WIDE] = jnp.zeros((FUSE_ROWS, WIDE - DIM), jnp.float32)


def _fused_table(emb_weight, lora_At, lora_B):
    grid = NUM_EMB // FUSE_ROWS
    return pl.pallas_call(
        _fuse_body,
        grid=(grid,),
        in_specs=[
            pl.BlockSpec((FUSE_ROWS, RANK), lambda i: (i, 0)),
            pl.BlockSpec((FUSE_ROWS, DIM), lambda i: (i, 0)),
            pl.BlockSpec((DIM, RANK), lambda i: (0, 0)),
        ],
        out_specs=pl.BlockSpec((FUSE_ROWS, WIDE), lambda i: (i, 0)),
        out_shape=jax.ShapeDtypeStruct((NUM_EMB, WIDE), jnp.float32),
    )(lora_At, emb_weight, lora_B)


NUM_CORES = 2
NUM_SUBCORES = 16
NW = NUM_CORES * NUM_SUBCORES
BATCH = 4096
SEQ = 200
NSPAN = 8
SPAN = BATCH // NSPAN
NTS = 4
TPW = SEQ // NTS
HALF = SPAN // 2
NU = TPW * 2
L = 16
TRP = 261


def _gather_body(table_hbm, xt_hbm, out_hbm, idx_v, rows_v, tr_v, gsem, osem):
    wid = lax.axis_index("c") * NUM_SUBCORES + lax.axis_index("s")
    q = wid // NTS
    rid = wid % NTS
    b0 = q * SPAN

    def stage_idx(j):
        pltpu.sync_copy(
            xt_hbm.at[rid + NTS * j, pl.ds(b0, SPAN)], idx_v.at[lax.rem(j, 2)]
        )

    stage_idx(0)

    iota = lax.iota(jnp.int32, L)
    zero = iota - iota
    dim_sel = [iota + (k2 * L) for k2 in range(DIM // L)]

    def gather_descs(u, slot):
        j = u // 2
        h = u % 2
        return [
            pltpu.make_async_copy(
                table_hbm.at[
                    idx_v.at[lax.rem(j, 2), pl.ds(h * HALF + c * 128, 128)]
                ],
                rows_v.at[slot, pl.ds(c * 128, 128)],
                gsem.at[slot],
            )
            for c in range(2)
        ]

    def out_desc(u, slot):
        j = u // 2
        h = u % 2
        return pltpu.make_async_copy(
            tr_v.at[slot, :, pl.ds(0, HALF)],
            out_hbm.at[rid + NTS * j, :, pl.ds(b0 + h * HALF, HALF)],
            osem.at[slot],
        )

    for d in gather_descs(0, 0):
        d.start()

    def step(u, carry):
        slot = lax.rem(u, 2)

        @pl.when((u < NU - 1) & (lax.rem(u, 2) == 1))
        def _():
            stage_idx(u // 2 + 1)

        @pl.when(u < NU - 1)
        def _():
            for d in gather_descs(u + 1, 1 - slot):
                d.start()

        for d in gather_descs(u, slot):
            d.wait()

        @pl.when(u >= 2)
        def _():
            out_desc(u - 2, slot).wait()

        @plsc.parallel_loop(0, HALF, unroll=8)
        def trans_blk(b2):
            bcol = zero + b2
            for k2 in range(DIM // L):
                v = rows_v[slot, b2, pl.ds(k2 * L, L)]
                plsc.store_scatter(tr_v.at[slot], [dim_sel[k2], bcol], v)

        out_desc(u, slot).start()
        return carry

    lax.fori_loop(0, NU, step, 0)
    out_desc(NU - 2, 0).wait()
    out_desc(NU - 1, 1).wait()


def _gather(table, xt):
    mesh = plsc.VectorSubcoreMesh(core_axis_name="c", subcore_axis_name="s")
    run = functools.partial(
        pl.kernel,
        mesh=mesh,
        out_type=jax.ShapeDtypeStruct((SEQ, DIM, BATCH), jnp.float32),
        scratch_types=[
            pltpu.VMEM((2, SPAN), jnp.int32),
            pltpu.VMEM((2, HALF, WIDE), jnp.float32),
            pltpu.VMEM((2, DIM, TRP), jnp.float32),
            pltpu.SemaphoreType.DMA((2,)),
            pltpu.SemaphoreType.DMA((2,)),
        ],
        compiler_params=pltpu.CompilerParams(needs_layout_passes=False),
    )(_gather_body)
    return run(table, xt)


def kernel(x, emb_weight, lora_A, lora_B):
    table = _fused_table(emb_weight, lora_A.T, lora_B)
    out_t = _gather(table, x.T.astype(jnp.int32))
    return jnp.transpose(out_t, (2, 0, 1))


# speedup vs baseline: 2.3137x; 2.2656x over previous
import functools

import jax
import jax.numpy as jnp
from jax import lax
from jax.experimental import pallas as pl
from jax.experimental.pallas import tpu as pltpu
from jax.experimental.pallas import tpu_sc as plsc

NUM_EMB = 100000
DIM = 64
WIDE = 128
RANK = 8
SCALE = 2.0


FUSE_ROWS = 2000


def _fuse_body(a_ref, emb_ref, b_ref, out_ref):
    delta = lax.dot_general(
        a_ref[...], b_ref[...],
        dimension_numbers=(((1,), (1,)), ((), ())),
        preferred_element_type=jnp.float32,
    )
    out_ref[:, 0:DIM] = emb_ref[...] + SCALE * delta
    out_ref[:, DIM:WIDE] = jnp.zeros((FUSE_ROWS, WIDE - DIM), jnp.float32)


def _fused_table(emb_weight, lora_At, lora_B):
    grid = NUM_EMB // FUSE_ROWS
    return pl.pallas_call(
        _fuse_body,
        grid=(grid,),
        in_specs=[
            pl.BlockSpec((FUSE_ROWS, RANK), lambda i: (i, 0)),
            pl.BlockSpec((FUSE_ROWS, DIM), lambda i: (i, 0)),
            pl.BlockSpec((DIM, RANK), lambda i: (0, 0)),
        ],
        out_specs=pl.BlockSpec((FUSE_ROWS, WIDE), lambda i: (i, 0)),
        out_shape=jax.ShapeDtypeStruct((NUM_EMB, WIDE), jnp.float32),
    )(lora_At, emb_weight, lora_B)


NUM_CORES = 2
NUM_SUBCORES = 16
NW = NUM_CORES * NUM_SUBCORES
BATCH = 4096
SEQ = 200
NSPAN = 8
SPAN = BATCH // NSPAN
NTS = 4
TPW = SEQ // NTS
HALF = SPAN // 2
NU = TPW * 2
L = 16


def _gather_body(table_hbm, xt_hbm, out_hbm, idx_v, rows_v, tr_v, gsem, osem):
    wid = lax.axis_index("c") * NUM_SUBCORES + lax.axis_index("s")
    q = wid // NTS
    rid = wid % NTS
    b0 = q * SPAN

    def stage_idx(j):
        pltpu.sync_copy(
            xt_hbm.at[rid + NTS * j, pl.ds(b0, SPAN)], idx_v.at[lax.rem(j, 2)]
        )

    stage_idx(0)

    iota = lax.iota(jnp.int32, L)

    def gather_descs(u, slot):
        j = u // 2
        h = u % 2
        return [
            pltpu.make_async_copy(
                table_hbm.at[
                    idx_v.at[lax.rem(j, 2), pl.ds(h * HALF + c * 128, 128)]
                ],
                rows_v.at[slot, pl.ds(c * 128, 128)],
                gsem.at[slot],
            )
            for c in range(2)
        ]

    def out_desc(u, slot):
        j = u // 2
        h = u % 2
        return pltpu.make_async_copy(
            tr_v.at[slot],
            out_hbm.at[rid + NTS * j, :, pl.ds(b0 + h * HALF, HALF)],
            osem.at[slot],
        )

    for d in gather_descs(0, 0):
        d.start()

    def step(u, carry):
        slot = lax.rem(u, 2)

        @pl.when((u < NU - 1) & (lax.rem(u, 2) == 1))
        def _():
            stage_idx(u // 2 + 1)

        @pl.when(u < NU - 1)
        def _():
            for d in gather_descs(u + 1, 1 - slot):
                d.start()

        for d in gather_descs(u, slot):
            d.wait()

        @pl.when(u >= 2)
        def _():
            out_desc(u - 2, slot).wait()

        @plsc.parallel_loop(0, HALF // L, unroll=2)
        def trans_blk(blk):
            brow = iota + blk * L
            for k2 in range(DIM // L):
                for m in range(L):
                    dcol = ((iota + m) & (L - 1)) + k2 * L
                    v = plsc.load_gather(rows_v.at[slot], [brow, dcol])
                    plsc.store_scatter(tr_v.at[slot], [dcol, brow], v)

        out_desc(u, slot).start()
        return carry

    lax.fori_loop(0, NU, step, 0)
    out_desc(NU - 2, 0).wait()
    out_desc(NU - 1, 1).wait()


def _gather(table, xt):
    mesh = plsc.VectorSubcoreMesh(core_axis_name="c", subcore_axis_name="s")
    run = functools.partial(
        pl.kernel,
        mesh=mesh,
        out_type=jax.ShapeDtypeStruct((SEQ, DIM, BATCH), jnp.float32),
        scratch_types=[
            pltpu.VMEM((2, SPAN), jnp.int32),
            pltpu.VMEM((2, HALF, WIDE), jnp.float32),
            pltpu.VMEM((2, DIM, HALF), jnp.float32),
            pltpu.SemaphoreType.DMA((2,)),
            pltpu.SemaphoreType.DMA((2,)),
        ],
        compiler_params=pltpu.CompilerParams(needs_layout_passes=False),
    )(_gather_body)
    return run(table, xt)


def kernel(x, emb_weight, lora_A, lora_B):
    table = _fused_table(emb_weight, lora_A.T, lora_B)
    out_t = _gather(table, x.T.astype(jnp.int32))
    return jnp.transpose(out_t, (2, 0, 1))
